# baseline (speedup 1.0000x reference)
import jax
import jax.numpy as jnp
from jax.experimental import pallas as pl
from jax.experimental.pallas import tpu as pltpu

NORM_PERCENT = 50
CB = 16


def _sumsq_body(x_ref, out_ref):
    xb = x_ref[...]
    out_ref[...] = jnp.sum(xb * xb, axis=1).reshape(1, 1, -1)


def _plan_body(k, s_ref, plan_ref):
    s = s_ref[0, :]
    n = s.shape[0]
    a = jax.lax.broadcast_in_dim(s, (n, n), (0,))
    b = jax.lax.broadcast_in_dim(s, (n, n), (1,))
    jidx = jax.lax.broadcasted_iota(jnp.int32, (n, n), 0)
    cidx = jax.lax.broadcasted_iota(jnp.int32, (n, n), 1)
    beats = (a > b) | ((a == b) & (jidx < cidx))
    rank = jnp.sum(beats.astype(jnp.int32), axis=0)
    plan_ref[0, :] = (rank < k).astype(jnp.int32)


def _mul_body(plan_ref, x_hbm, o_hbm, xbuf, rsems, wsems):
    b = pl.program_id(0)
    nb = pl.num_programs(0)

    def rd(c, sl, ch):
        return pltpu.make_async_copy(
            x_hbm.at[pl.ds(c, 1)], xbuf.at[sl, pl.ds(ch, 1)], rsems.at[sl]
        )

    def reads(bb, sl, action):
        for ch in range(CB):
            c = bb * CB + ch

            @pl.when(plan_ref[0, c] == 1)
            def _():
                action(rd(c, sl, ch))

    def writes(bb, sl, action):
        action(pltpu.make_async_copy(
            xbuf.at[sl], o_hbm.at[pl.ds(bb * CB, CB)], wsems.at[sl],
        ))

    @pl.when(b == 0)
    def _():
        reads(0, 0, lambda cp: cp.start())

    @pl.when(b > 0)
    def _():
        writes(b - 1, (b - 1) % 2, lambda cp: cp.wait())

    @pl.when(b + 1 < nb)
    def _():
        reads(b + 1, (b + 1) % 2, lambda cp: cp.start())

    slot_ref = xbuf.at[b % 2]
    for ch in range(CB):
        @pl.when(plan_ref[0, b * CB + ch] == 0)
        def _():
            slot_ref[pl.ds(ch, 1), :] = jnp.zeros_like(slot_ref[pl.ds(ch, 1), :])

    reads(b, b % 2, lambda cp: cp.wait())
    writes(b, b % 2, lambda cp: cp.start())

    @pl.when(b == nb - 1)
    def _():
        writes(b, b % 2, lambda cp: cp.wait())


def kernel(input):
    x = input
    _, C, H, W = x.shape
    k = int(float(NORM_PERCENT) / 100.0 * float(C))
    if k <= 0 or k >= C:
        k = C
    HW = H * W
    nblk = C // CB

    x2 = x.reshape(C, HW)

    sumsq = pl.pallas_call(
        _sumsq_body,
        grid=(nblk,),
        in_specs=[pl.BlockSpec((CB, HW), lambda i: (i, 0))],
        out_specs=pl.BlockSpec((1, 1, CB), lambda i: (i, 0, 0)),
        out_shape=jax.ShapeDtypeStruct((nblk, 1, CB), jnp.float32),
    )(x2)

    plan = pl.pallas_call(
        lambda s_ref, plan_ref: _plan_body(k, s_ref, plan_ref),
        in_specs=[pl.BlockSpec((1, C), lambda: (0, 0))],
        out_specs=pl.BlockSpec((1, C), lambda: (0, 0)),
        out_shape=jax.ShapeDtypeStruct((1, C), jnp.int32),
    )(sumsq.reshape(1, C))

    grid_spec = pltpu.PrefetchScalarGridSpec(
        num_scalar_prefetch=1,
        grid=(nblk,),
        in_specs=[pl.BlockSpec(memory_space=pl.ANY)],
        out_specs=pl.BlockSpec(memory_space=pl.ANY),
        scratch_shapes=[
            pltpu.VMEM((2, CB, HW), jnp.float32),
            pltpu.SemaphoreType.DMA((2,)),
            pltpu.SemaphoreType.DMA((2,)),
        ],
    )
    out = pl.pallas_call(
        _mul_body,
        grid_spec=grid_spec,
        out_shape=jax.ShapeDtypeStruct((C, HW), jnp.float32),
    )(plan, x2)

    return out.reshape(x.shape)


# speedup vs baseline: 1.1182x; 1.1182x over previous
import jax
import jax.numpy as jnp
from jax.experimental import pallas as pl
from jax.experimental.pallas import tpu as pltpu

NORM_PERCENT = 50
CB = 32


def _sumsq_body(x_ref, out_ref):
    xb = x_ref[...]
    out_ref[...] = jnp.sum(xb * xb, axis=1).reshape(1, 1, -1)


def _plan_body(k, s_ref, plan_ref):
    s = s_ref[0, :]
    n = s.shape[0]
    a = jax.lax.broadcast_in_dim(s, (n, n), (0,))
    b = jax.lax.broadcast_in_dim(s, (n, n), (1,))
    jidx = jax.lax.broadcasted_iota(jnp.int32, (n, n), 0)
    cidx = jax.lax.broadcasted_iota(jnp.int32, (n, n), 1)
    beats = (a > b) | ((a == b) & (jidx < cidx))
    rank = jnp.sum(beats.astype(jnp.int32), axis=0)
    plan_ref[0, :] = (rank < k).astype(jnp.int32)


def _mul_body(plan_ref, x_hbm, o_hbm, xbuf, zbuf, rsems, wsems):
    b = pl.program_id(0)
    nb = pl.num_programs(0)

    def rd(c, sl, ch):
        return pltpu.make_async_copy(
            x_hbm.at[pl.ds(c, 1)], xbuf.at[sl, pl.ds(ch, 1)], rsems.at[sl]
        )

    def reads(bb, sl, action):
        for ch in range(CB):
            c = bb * CB + ch

            @pl.when(plan_ref[0, c] == 1)
            def _():
                action(rd(c, sl, ch))

    def writes(bb, sl, action):
        for ch in range(CB):
            c = bb * CB + ch

            @pl.when(plan_ref[0, c] == 1)
            def _():
                action(pltpu.make_async_copy(
                    xbuf.at[sl, pl.ds(ch, 1)], o_hbm.at[pl.ds(c, 1)],
                    wsems.at[sl],
                ))

            @pl.when(plan_ref[0, c] == 0)
            def _():
                action(pltpu.make_async_copy(
                    zbuf, o_hbm.at[pl.ds(c, 1)], wsems.at[sl],
                ))

    @pl.when(b == 0)
    def _():
        zbuf[...] = jnp.zeros_like(zbuf)
        reads(0, 0, lambda cp: cp.start())

    @pl.when(b > 0)
    def _():
        writes(b - 1, (b - 1) % 2, lambda cp: cp.wait())

    @pl.when(b + 1 < nb)
    def _():
        reads(b + 1, (b + 1) % 2, lambda cp: cp.start())

    reads(b, b % 2, lambda cp: cp.wait())
    writes(b, b % 2, lambda cp: cp.start())

    @pl.when(b == nb - 1)
    def _():
        writes(b, b % 2, lambda cp: cp.wait())


def kernel(input):
    x = input
    _, C, H, W = x.shape
    k = int(float(NORM_PERCENT) / 100.0 * float(C))
    if k <= 0 or k >= C:
        k = C
    HW = H * W
    nblk = C // CB

    x2 = x.reshape(C, HW)

    sumsq = pl.pallas_call(
        _sumsq_body,
        grid=(nblk,),
        in_specs=[pl.BlockSpec((CB, HW), lambda i: (i, 0))],
        out_specs=pl.BlockSpec((1, 1, CB), lambda i: (i, 0, 0)),
        out_shape=jax.ShapeDtypeStruct((nblk, 1, CB), jnp.float32),
    )(x2)

    plan = pl.pallas_call(
        lambda s_ref, plan_ref: _plan_body(k, s_ref, plan_ref),
        in_specs=[pl.BlockSpec((1, C), lambda: (0, 0))],
        out_specs=pl.BlockSpec((1, C), lambda: (0, 0)),
        out_shape=jax.ShapeDtypeStruct((1, C), jnp.int32),
    )(sumsq.reshape(1, C))

    grid_spec = pltpu.PrefetchScalarGridSpec(
        num_scalar_prefetch=1,
        grid=(nblk,),
        in_specs=[pl.BlockSpec(memory_space=pl.ANY)],
        out_specs=pl.BlockSpec(memory_space=pl.ANY),
        scratch_shapes=[
            pltpu.VMEM((2, CB, HW), jnp.float32),
            pltpu.VMEM((1, HW), jnp.float32),
            pltpu.SemaphoreType.DMA((2,)),
            pltpu.SemaphoreType.DMA((2,)),
        ],
    )
    out = pl.pallas_call(
        _mul_body,
        grid_spec=grid_spec,
        out_shape=jax.ShapeDtypeStruct((C, HW), jnp.float32),
    )(plan, x2)

    return out.reshape(x.shape)
